# Initial kernel scaffold; baseline (speedup 1.0000x reference)
import jax
import jax.numpy as jnp
from jax.experimental import pallas as pl


def kernel(h, edge_index, W0, b0, g0, be0, W1, b1, g1, be1, W2, b2, g2, be2, Wc, bc, gc, bec):
    raise NotImplementedError("write your pallas kernel here")


# speedup vs baseline: 2.5763x; 2.5763x over previous
import functools

import jax
import jax.numpy as jnp
from jax import lax
from jax.experimental import pallas as pl
from jax.experimental.pallas import tpu as pltpu
from jax.experimental.pallas import tpu_sc as plsc

N = 10000
E = 320000
D = 128

NC = 2
NS = 16
NW = NC * NS

NPAD = 10240
EPW = 10240
EPAD = EPW * NW
C = 128
NCHUNK = EPW // C
ROWS_PER_TILE = NPAD // NS
L = 16

_mesh = plsc.VectorSubcoreMesh(core_axis_name="c", subcore_axis_name="s")


def _make_sc_agg(with_deg):
    out_type = [jax.ShapeDtypeStruct((NC * NPAD, D), jnp.float32)]
    if with_deg:
        out_type.append(jax.ShapeDtypeStruct((NW, NPAD), jnp.float32))

    def body(src_hbm, dst_hbm, feat_hbm, zero_hbm, *rest):
        if with_deg:
            acc_out, deg_out, srcv, dstv, rowsv, degv, acc_sh, sem = rest
        else:
            acc_out, srcv, dstv, rowsv, degv, acc_sh, sem = rest
        cid = lax.axis_index("c")
        sid = lax.axis_index("s")
        wid = cid * NS + sid

        r0 = sid * ROWS_PER_TILE
        pltpu.sync_copy(zero_hbm.at[pl.ds(r0, ROWS_PER_TILE)],
                        acc_sh.at[pl.ds(r0, ROWS_PER_TILE)])
        if with_deg:
            def zbody(i, carry):
                degv[pl.ds(i * L, L)] = jnp.zeros((L,), jnp.float32)
                return carry
            lax.fori_loop(0, NPAD // L, zbody, 0)
        plsc.subcore_barrier()

        ebase = wid * EPW
        ones = jnp.ones((L,), jnp.float32)

        def body_g(g, carry):
            off = ebase + g * C
            pltpu.sync_copy(src_hbm.at[pl.ds(off, C)], srcv)
            pltpu.sync_copy(dst_hbm.at[pl.ds(off, C)], dstv)
            pltpu.async_copy(feat_hbm.at[srcv], rowsv, sem).wait()
            pltpu.sync_copy(rowsv, acc_sh.at[dstv], add=True)
            if with_deg:
                for j in range(C // L):
                    idx = dstv[pl.ds(j * L, L)]
                    plsc.addupdate_scatter(degv, [idx], ones)
            return carry

        lax.fori_loop(0, NCHUNK, body_g, 0)

        plsc.subcore_barrier()
        pltpu.sync_copy(acc_sh.at[pl.ds(r0, ROWS_PER_TILE)],
                        acc_out.at[pl.ds(cid * NPAD + r0, ROWS_PER_TILE)])
        if with_deg:
            pltpu.sync_copy(degv, deg_out.at[wid])

    scratch = [
        pltpu.VMEM((C,), jnp.int32),
        pltpu.VMEM((C,), jnp.int32),
        pltpu.VMEM((C, D), jnp.float32),
        pltpu.VMEM((NPAD,), jnp.float32),
        pltpu.VMEM_SHARED((NPAD, D), jnp.float32),
        pltpu.SemaphoreType.DMA,
    ]
    return pl.kernel(body, out_type=out_type, mesh=_mesh,
                     scratch_types=scratch,
                     compiler_params=pltpu.CompilerParams(
                         needs_layout_passes=False))


_sc_agg_deg = _make_sc_agg(True)
_sc_agg = _make_sc_agg(False)


def _bn_relu(z, gamma, beta, eps=1e-5):
    mu = jnp.mean(z, axis=0)
    var = jnp.mean(jnp.square(z - mu), axis=0)
    return jnp.maximum(gamma * (z - mu) * lax.rsqrt(var + eps) + beta, 0.0)


def _combine(acc_ref, inv_deg):
    acc = acc_ref[0:NPAD, :] + acc_ref[NPAD:2 * NPAD, :]
    return acc[:N, :] * inv_deg


def _tc_stage1(acc_ref, degp_ref, w0, b0, g0, be0, w1, b1, g1, be1,
               s1pad_ref, t1_ref, invdeg_ref):
    deg = jnp.sum(degp_ref[...], axis=0)[:N]
    inv_deg = (1.0 / jnp.clip(deg, 1.0, None))[:, None]
    a0 = _combine(acc_ref, inv_deg)
    z0 = jnp.dot(a0, w0[...], preferred_element_type=jnp.float32) + b0[...]
    s1 = _bn_relu(z0, g0[...], be0[...])
    z1 = jnp.dot(a0, w1[...], preferred_element_type=jnp.float32) + b1[...]
    t1_ref[...] = _bn_relu(z1, g1[...], be1[...])
    s1pad_ref[...] = jnp.concatenate(
        [s1, jnp.zeros((NPAD - N, D), jnp.float32)], axis=0)
    invdeg_ref[...] = inv_deg


def _tc_stage2(acc_ref, invdeg_ref, s1pad_ref, t1_ref, h_ref,
               w2, b2, g2, be2, wc, bc, gc, bec, out_ref):
    a1 = _combine(acc_ref, invdeg_ref[...])
    z2 = jnp.dot(a1, w2[...], preferred_element_type=jnp.float32) + b2[...]
    s2 = t1_ref[...] + _bn_relu(z2, g2[...], be2[...])
    s1 = s1pad_ref[0:N, :]
    zc = (jnp.dot(s1, wc[0:D, :], preferred_element_type=jnp.float32)
          + jnp.dot(s2, wc[D:2 * D, :], preferred_element_type=jnp.float32)
          + bc[...])
    out_ref[...] = h_ref[...] + _bn_relu(zc, gc[...], bec[...])


def kernel(h, edge_index, W0, b0, g0, be0, W1, b1, g1, be1,
           W2, b2, g2, be2, Wc, bc, gc, bec):
    src = edge_index[0]
    dst = edge_index[1]
    pad = EPAD - E
    src_pad = jnp.concatenate([src, jnp.zeros((pad,), jnp.int32)])
    dst_pad = jnp.concatenate([dst, jnp.full((pad,), N, jnp.int32)])
    hpad = jnp.concatenate(
        [h, jnp.zeros((NPAD - N, D), jnp.float32)], axis=0)
    zeros_init = jnp.zeros((NPAD, D), jnp.float32)

    acc_a, deg_p = _sc_agg_deg(src_pad, dst_pad, hpad, zeros_init)

    s1pad, t1, inv_deg = pl.pallas_call(
        _tc_stage1,
        out_shape=[
            jax.ShapeDtypeStruct((NPAD, D), jnp.float32),
            jax.ShapeDtypeStruct((N, D), jnp.float32),
            jax.ShapeDtypeStruct((N, 1), jnp.float32),
        ],
    )(acc_a, deg_p, W0, b0, g0, be0, W1, b1, g1, be1)

    acc_b = _sc_agg(src_pad, dst_pad, s1pad, zeros_init)[0]

    out = pl.pallas_call(
        _tc_stage2,
        out_shape=jax.ShapeDtypeStruct((N, D), jnp.float32),
    )(acc_b, inv_deg, s1pad, t1, h, W2, b2, g2, be2, Wc, bc, gc, bec)
    return out


# speedup vs baseline: 3.5375x; 1.3731x over previous
import functools

import jax
import jax.numpy as jnp
from jax import lax
from jax.experimental import pallas as pl
from jax.experimental.pallas import tpu as pltpu
from jax.experimental.pallas import tpu_sc as plsc

N = 10000
E = 320000
D = 128

NC = 2
NS = 16
NW = NC * NS

NPAD = 10240
EPW = 10240
EPAD = EPW * NW
C = 128
NCHUNK = EPW // C
ROWS_PER_TILE = NPAD // NS
L = 16
NB = 2

_mesh = plsc.VectorSubcoreMesh(core_axis_name="c", subcore_axis_name="s")


def _make_sc_agg(with_deg):
    out_type = [jax.ShapeDtypeStruct((NC * NPAD, D), jnp.float32)]
    if with_deg:
        out_type.append(jax.ShapeDtypeStruct((NW, NPAD), jnp.float32))

    def body(idx_hbm, feat_hbm, zero_hbm, *rest):
        if with_deg:
            acc_out, deg_out, idxv, rowsv, degv, acc_sh, *sems = rest
        else:
            degv = None
            acc_out, idxv, rowsv, acc_sh, *sems = rest
        isem = sems[0:NB]
        gsem = sems[NB:2 * NB]
        ssem = sems[2 * NB:3 * NB]
        cid = lax.axis_index("c")
        sid = lax.axis_index("s")
        wid = cid * NS + sid

        for b in range(NB):
            pltpu.async_copy(idx_hbm.at[wid, b], idxv.at[b], isem[b])

        r0 = sid * ROWS_PER_TILE
        pltpu.sync_copy(zero_hbm.at[pl.ds(r0, ROWS_PER_TILE)],
                        acc_sh.at[pl.ds(r0, ROWS_PER_TILE)])
        if with_deg:
            def zbody(i, carry):
                degv[pl.ds(i * L, L)] = jnp.zeros((L,), jnp.float32)
                return carry
            lax.fori_loop(0, NPAD // L, zbody, 0)
        plsc.subcore_barrier()

        ones = jnp.ones((L,), jnp.float32)

        def body_k(k, carry):
            for b in range(NB):
                g = k * NB + b
                pltpu.make_async_copy(
                    idx_hbm.at[wid, g], idxv.at[b], isem[b]).wait()
                pltpu.async_copy(
                    feat_hbm.at[idxv.at[b, 0]], rowsv.at[b], gsem[b])
            for b in range(NB):
                if with_deg:
                    for j in range(C // L):
                        idx = idxv[b, 1, pl.ds(j * L, L)]
                        plsc.addupdate_scatter(degv, [idx], ones)
                pltpu.make_async_copy(
                    feat_hbm.at[idxv.at[b, 0]], rowsv.at[b], gsem[b]).wait()
                pltpu.async_copy(
                    rowsv.at[b], acc_sh.at[idxv.at[b, 1]], ssem[b], add=True)
            for b in range(NB):
                g = k * NB + b
                pltpu.make_async_copy(
                    rowsv.at[b], acc_sh.at[idxv.at[b, 1]], ssem[b]).wait()
                gn = g + NB

                @pl.when(gn < NCHUNK)
                def _():
                    pltpu.async_copy(
                        idx_hbm.at[wid, gn], idxv.at[b], isem[b])
            return carry

        lax.fori_loop(0, NCHUNK // NB, body_k, 0)

        plsc.subcore_barrier()
        pltpu.sync_copy(acc_sh.at[pl.ds(r0, ROWS_PER_TILE)],
                        acc_out.at[pl.ds(cid * NPAD + r0, ROWS_PER_TILE)])
        if with_deg:
            pltpu.sync_copy(degv, deg_out.at[wid])

    scratch = [
        pltpu.VMEM((NB, 2, C), jnp.int32),
        pltpu.VMEM((NB, C, D), jnp.float32),
    ]
    if with_deg:
        scratch.append(pltpu.VMEM((NPAD,), jnp.float32))
    scratch.append(pltpu.VMEM_SHARED((NPAD, D), jnp.float32))
    scratch += [pltpu.SemaphoreType.DMA] * (3 * NB)
    return pl.kernel(body, out_type=out_type, mesh=_mesh,
                     scratch_types=scratch,
                     compiler_params=pltpu.CompilerParams(
                         needs_layout_passes=False))


_sc_agg_deg = _make_sc_agg(True)
_sc_agg = _make_sc_agg(False)


def _bn_relu(z, gamma, beta, eps=1e-5):
    mu = jnp.mean(z, axis=0)
    var = jnp.mean(jnp.square(z - mu), axis=0)
    return jnp.maximum(gamma * (z - mu) * lax.rsqrt(var + eps) + beta, 0.0)


def _combine(acc_ref, inv_deg):
    acc = acc_ref[0:NPAD, :] + acc_ref[NPAD:2 * NPAD, :]
    return acc[:N, :] * inv_deg


def _tc_stage1(acc_ref, degp_ref, w0, b0, g0, be0, w1, b1, g1, be1,
               s1pad_ref, t1_ref, invdeg_ref):
    deg = jnp.sum(degp_ref[...], axis=0)[:N]
    inv_deg = (1.0 / jnp.clip(deg, 1.0, None))[:, None]
    a0 = _combine(acc_ref, inv_deg)
    z0 = jnp.dot(a0, w0[...], preferred_element_type=jnp.float32) + b0[...]
    s1 = _bn_relu(z0, g0[...], be0[...])
    z1 = jnp.dot(a0, w1[...], preferred_element_type=jnp.float32) + b1[...]
    t1_ref[...] = _bn_relu(z1, g1[...], be1[...])
    s1pad_ref[...] = jnp.concatenate(
        [s1, jnp.zeros((NPAD - N, D), jnp.float32)], axis=0)
    invdeg_ref[...] = inv_deg


def _tc_stage2(acc_ref, invdeg_ref, s1pad_ref, t1_ref, h_ref,
               w2, b2, g2, be2, wc, bc, gc, bec, out_ref):
    a1 = _combine(acc_ref, invdeg_ref[...])
    z2 = jnp.dot(a1, w2[...], preferred_element_type=jnp.float32) + b2[...]
    s2 = t1_ref[...] + _bn_relu(z2, g2[...], be2[...])
    s1 = s1pad_ref[0:N, :]
    zc = (jnp.dot(s1, wc[0:D, :], preferred_element_type=jnp.float32)
          + jnp.dot(s2, wc[D:2 * D, :], preferred_element_type=jnp.float32)
          + bc[...])
    out_ref[...] = h_ref[...] + _bn_relu(zc, gc[...], bec[...])


def kernel(h, edge_index, W0, b0, g0, be0, W1, b1, g1, be1,
           W2, b2, g2, be2, Wc, bc, gc, bec):
    src = edge_index[0]
    dst = edge_index[1]
    pad = EPAD - E
    src_pad = jnp.concatenate([src, jnp.zeros((pad,), jnp.int32)])
    dst_pad = jnp.concatenate([dst, jnp.full((pad,), N, jnp.int32)])
    idx4 = jnp.stack([src_pad.reshape(NW, NCHUNK, C),
                      dst_pad.reshape(NW, NCHUNK, C)], axis=2)
    hpad = jnp.concatenate(
        [h, jnp.zeros((NPAD - N, D), jnp.float32)], axis=0)
    zeros_init = jnp.zeros((NPAD, D), jnp.float32)

    acc_a, deg_p = _sc_agg_deg(idx4, hpad, zeros_init)

    s1pad, t1, inv_deg = pl.pallas_call(
        _tc_stage1,
        out_shape=[
            jax.ShapeDtypeStruct((NPAD, D), jnp.float32),
            jax.ShapeDtypeStruct((N, D), jnp.float32),
            jax.ShapeDtypeStruct((N, 1), jnp.float32),
        ],
    )(acc_a, deg_p, W0, b0, g0, be0, W1, b1, g1, be1)

    acc_b = _sc_agg(idx4, s1pad, zeros_init)[0]

    out = pl.pallas_call(
        _tc_stage2,
        out_shape=jax.ShapeDtypeStruct((N, D), jnp.float32),
    )(acc_b, inv_deg, s1pad, t1, h, W2, b2, g2, be2, Wc, bc, gc, bec)
    return out
